# Initial kernel scaffold; baseline (speedup 1.0000x reference)
import jax
import jax.numpy as jnp
from jax.experimental import pallas as pl


def kernel(inputs, lt):
    raise NotImplementedError("write your pallas kernel here")


# speedup vs baseline: 7.6561x; 7.6561x over previous
import functools

import jax
import jax.numpy as jnp
from jax import lax
from jax.experimental import pallas as pl
from jax.experimental.pallas import tpu as pltpu
from jax.experimental.pallas import tpu_sc as plsc

EPS = 1e-5
BOUNDARY = 1.0 - EPS
VOCAB = 100000
DIM = 128
BATCH = 4096
NSAMPLES = 51

NC, NS, LANES = 2, 16, 16
NW = NC * NS
BPW = BATCH // NW
NCHUNK = DIM // LANES


NPAD = 64
NGROUP = NPAD // LANES


def _sc_reduce(idx, lt):
    mesh = plsc.VectorSubcoreMesh(
        core_axis_name="c", subcore_axis_name="s", num_cores=NC, num_subcores=NS
    )

    @functools.partial(
        pl.kernel,
        out_type=[
            jax.ShapeDtypeStruct((BATCH, NPAD), jnp.float32),
            jax.ShapeDtypeStruct((BATCH, NPAD), jnp.float32),
        ],
        mesh=mesh,
        scratch_types=[
            pltpu.VMEM((BPW, NSAMPLES), jnp.int32),
            pltpu.VMEM((NSAMPLES, DIM), jnp.float32),
            pltpu.VMEM((NSAMPLES, DIM), jnp.float32),
            pltpu.VMEM((BPW, NPAD), jnp.float32),
            pltpu.VMEM((BPW, NPAD), jnp.float32),
            pltpu.SemaphoreType.DMA,
            pltpu.SemaphoreType.DMA,
        ],
    )
    def kern(lt_hbm, idx_hbm, sq_hbm, uv_hbm,
             idx_v, rows_a, rows_b, sq_v, uv_v, sem_a, sem_b):
        wid = lax.axis_index("s") * NC + lax.axis_index("c")
        base = wid * BPW
        pltpu.sync_copy(idx_hbm.at[pl.ds(base, BPW)], idx_v)

        bufs = (rows_a, rows_b)
        sems = (sem_a, sem_b)

        pltpu.async_copy(lt_hbm.at[idx_v.at[0]], rows_a, sem_a)
        pltpu.async_copy(lt_hbm.at[idx_v.at[1]], rows_b, sem_b)

        lane = lax.iota(jnp.int32, LANES)

        def lane_sum(v):
            for sh in (8, 4, 2, 1):
                v = v + jnp.take(v, lane ^ sh)
            return v

        def compute(b, buf):
            u = [buf[0, pl.ds(c * LANES, LANES)] for c in range(NCHUNK)]

            @pl.loop(0, NGROUP)
            def _(g):
                sq_res = jnp.zeros((LANES,), jnp.float32)
                uv_res = jnp.zeros((LANES,), jnp.float32)
                g0 = g * LANES
                for l in range(LANES):
                    j = jnp.minimum(g0 + l, NSAMPLES - 1)
                    uv_acc = None
                    sq_acc = None
                    for c in range(NCHUNK):
                        v = buf[j, pl.ds(c * LANES, LANES)]
                        uvc = u[c] * v
                        sqc = v * v
                        uv_acc = uvc if uv_acc is None else uv_acc + uvc
                        sq_acc = sqc if sq_acc is None else sq_acc + sqc
                    onehot = lane == l
                    sq_res = jnp.where(onehot, lane_sum(sq_acc), sq_res)
                    uv_res = jnp.where(onehot, lane_sum(uv_acc), uv_res)
                sq_v[b, pl.ds(g0, LANES)] = sq_res
                uv_v[b, pl.ds(g0, LANES)] = uv_res

        @pl.loop(0, BPW, step=2)
        def _(b0):
            for t in range(2):
                b = b0 + t
                buf, sem = bufs[t], sems[t]
                pltpu.make_async_copy(lt_hbm.at[idx_v.at[b]], buf, sem).wait()
                compute(b, buf)

                @pl.when(b + 2 < BPW)
                def _():
                    pltpu.async_copy(lt_hbm.at[idx_v.at[b + 2]], buf, sem)

        pltpu.sync_copy(sq_v, sq_hbm.at[pl.ds(base, BPW)])
        pltpu.sync_copy(uv_v, uv_hbm.at[pl.ds(base, BPW)])

    return kern(lt, idx)


def _tc_finalize_body(sq_ref, uv_ref, out_ref):
    sq = sq_ref[...]
    squ = sq[:, 0:1]
    sqv = sq[:, 1:NSAMPLES]
    uv = uv_ref[:, 1:NSAMPLES]
    nu = jnp.sqrt(squ)
    nv = jnp.sqrt(sqv)
    fa = jnp.where(nu > BOUNDARY, BOUNDARY / jnp.maximum(nu, EPS), 1.0)
    fb = jnp.where(nv > BOUNDARY, BOUNDARY / jnp.maximum(nv, EPS), 1.0)
    squ_n = squ * (fa * fa)
    sqv_n = sqv * (fb * fb)
    uv_n = uv * (fa * fb)
    sqdist = squ_n + sqv_n - 2.0 * uv_n
    x = 1.0 + 2.0 * sqdist / ((1.0 - squ_n) * (1.0 - sqv_n))
    x = jnp.maximum(x, 1.0 + EPS)
    out_ref[...] = jnp.log(x + jnp.sqrt(x * x - 1.0))


def kernel(inputs, lt):
    idx = inputs.astype(jnp.int32)
    sq, uv = _sc_reduce(idx, lt)
    return pl.pallas_call(
        _tc_finalize_body,
        out_shape=jax.ShapeDtypeStruct((BATCH, NSAMPLES - 1), jnp.float32),
    )(sq, uv)
